# baseline (speedup 1.0000x reference)
import jax
import jax.numpy as jnp
from jax import lax
from jax.experimental import pallas as pl
from jax.experimental.pallas import tpu as pltpu
from jax.experimental.pallas import tpu_sc as plsc

VOCAB = 1_000_000
D = 64
T = 200
B = 4096
FLAT = B * T
NC = 2
NS = 16
NW = NC * NS
BPW = B // NW
NTB = T // 8
NBUF = 4
LANES = 16
NBG = BPW // LANES
KD = D // LANES


def _body(idxT_hbm, tok_hbm, pos_hbm, out_hbm,
          idxall, il0, il1, il2, il3, rows_v, ob0, ob1, posb,
          sg0, sg1, sg2, sg3, so0, so1):
    cid = lax.axis_index("c")
    sid = lax.axis_index("s")
    wid = sid * NC + cid
    il = [il0, il1, il2, il3]
    sg = [sg0, sg1, sg2, sg3]
    obuf = [ob0, ob1]
    so = [so0, so1]

    pltpu.sync_copy(idxT_hbm.at[:, pl.ds(wid * BPW, BPW)], idxall)

    def prep_idx(t, j):
        for v in range(NBG):
            sl = pl.ds(v * LANES, LANES)
            il[j][sl] = idxall[t, sl]

    def gather_start(j):
        pltpu.async_copy(tok_hbm.at[il[j]], rows_v.at[j], sg[j])

    def gather_wait(j):
        pltpu.make_async_copy(tok_hbm.at[pl.ds(0, BPW)], rows_v.at[j], sg[j]).wait()

    def out_start(t, ob):
        pltpu.async_copy(obuf[ob], out_hbm.at[t, :, wid], so[ob])

    def out_wait(t, ob):
        pltpu.make_async_copy(obuf[ob], out_hbm.at[t, :, wid], so[ob]).wait()

    rv = [jnp.int32(bg * LANES) + lax.iota(jnp.int32, LANES) for bg in range(NBG)]

    def transpose_add(tp, j, ob):
        def cpp_body(cpp, _):
            pk = posb[tp, pl.ds(cpp * LANES, LANES)]
            for cb2 in range(2):
                for ci in range(8):
                    lane = 8 * cb2 + ci
                    c = cpp * LANES + lane
                    pb = jax.lax.broadcast(pk[lane], (LANES,))
                    cvec = jax.lax.broadcast(c, (LANES,))
                    cbd = 2 * cpp + cb2
                    for bg in range(NBG):
                        val = plsc.load_gather(rows_v.at[j], [rv[bg], cvec])
                        obuf[ob][cbd, ci, pl.ds(bg * LANES, LANES)] = val + pb
            return ()

        lax.fori_loop(0, KD, cpp_body, ())

    for j in range(NBUF):
        prep_idx(j, j)
        gather_start(j)

    def tb_body(tb, _):
        t0 = tb * 8
        pltpu.sync_copy(pos_hbm.at[pl.ds(t0, 8)], posb)
        for tp in range(8):
            t = t0 + tp
            j = tp % NBUF
            ob = tp % 2
            gather_wait(j)
            if tp < 2:
                @pl.when(tb > 0)
                def _():
                    out_wait(t - 2, ob)
            else:
                out_wait(t - 2, ob)
            transpose_add(tp, j, ob)
            out_start(t, ob)

            @pl.when(t + NBUF < T)
            def _():
                prep_idx(t + NBUF, j)
                gather_start(j)
        return ()

    lax.fori_loop(0, NTB, tb_body, ())
    out_wait(T - 2, 0)
    out_wait(T - 1, 1)


@jax.jit
def _embed(idxT, tokp, posp):
    mesh = plsc.VectorSubcoreMesh(core_axis_name="c", subcore_axis_name="s")
    f = pl.kernel(
        _body,
        mesh=mesh,
        out_type=jax.ShapeDtypeStruct((T, 8, NW, 8, BPW), jnp.float32),
        scratch_types=[
            pltpu.VMEM((T, BPW), jnp.int32),
            pltpu.VMEM((BPW,), jnp.int32),
            pltpu.VMEM((BPW,), jnp.int32),
            pltpu.VMEM((BPW,), jnp.int32),
            pltpu.VMEM((BPW,), jnp.int32),
            pltpu.VMEM((NBUF, BPW, 2 * D), jnp.float32),
            pltpu.VMEM((8, 8, BPW), jnp.float32),
            pltpu.VMEM((8, 8, BPW), jnp.float32),
            pltpu.VMEM((8, 2 * D), jnp.float32),
            pltpu.SemaphoreType.DMA,
            pltpu.SemaphoreType.DMA,
            pltpu.SemaphoreType.DMA,
            pltpu.SemaphoreType.DMA,
            pltpu.SemaphoreType.DMA,
            pltpu.SemaphoreType.DMA,
        ],
        compiler_params=pltpu.CompilerParams(
            use_tc_tiling_on_sc=True, needs_layout_passes=False),
    )
    return f(idxT, tokp, posp)


def kernel(idx, token_embedding_table, position_embedding_table):
    idxT = idx.astype(jnp.int32).T
    tokp = jnp.pad(token_embedding_table, ((0, 0), (0, D)))
    posp = jnp.pad(position_embedding_table, ((0, 0), (0, D)))
    out = _embed(idxT, tokp, posp)
    return out.transpose(2, 4, 0, 1, 3).reshape(B, T, D)


# speedup vs baseline: 1.8919x; 1.8919x over previous
import jax
import jax.numpy as jnp
from jax import lax
from jax.experimental import pallas as pl
from jax.experimental.pallas import tpu as pltpu
from jax.experimental.pallas import tpu_sc as plsc

VOCAB = 1_000_000
D = 64
T = 200
B = 4096
FLAT = B * T
NC = 2
NS = 16
NW = NC * NS
PER_W = FLAT // NW
CHUNK = 128
NCHUNK = PER_W // CHUNK
NBUF = 4
NGRP = NCHUNK // NBUF
GROUP = NBUF * CHUNK
LANES = 16
KD = D // LANES


def _body(idx_hbm, tok_hbm, pos_hbm, out_hbm,
          ix0, ix1, rows_v, ob0, ob1, pat_v,
          sgi0, sgi1, sg0, sg1, sg2, sg3, so0, so1):
    cid = lax.axis_index("c")
    sid = lax.axis_index("s")
    wid = sid * NC + cid
    w0 = wid * PER_W
    ixg = [ix0, ix1]
    sgi = [sgi0, sgi1]
    sg = [sg0, sg1, sg2, sg3]
    so = [so0, so1]
    outb = [ob0, ob1]

    pltpu.sync_copy(pos_hbm, pat_v)

    def idx_start(g, p):
        pltpu.async_copy(idx_hbm.at[pl.ds(w0 + g * GROUP, GROUP)], ixg[p], sgi[p])

    def idx_wait(g, p):
        pltpu.make_async_copy(
            idx_hbm.at[pl.ds(w0 + g * GROUP, GROUP)], ixg[p], sgi[p]).wait()

    def gather_start(b, p):
        pltpu.async_copy(tok_hbm.at[ixg[p].at[pl.ds(b * CHUNK, CHUNK)]],
                         rows_v.at[b], sg[b])

    def gather_wait(b):
        pltpu.make_async_copy(tok_hbm.at[pl.ds(0, CHUNK)], rows_v.at[b], sg[b]).wait()

    def out_start(ci, ob):
        pltpu.async_copy(outb[ob], out_hbm.at[pl.ds(w0 + ci * CHUNK, CHUNK)], so[ob])

    def out_wait(ci, ob):
        pltpu.make_async_copy(
            outb[ob], out_hbm.at[pl.ds(w0 + ci * CHUNK, CHUNK)], so[ob]).wait()

    def extract_add(ci, b, ob):
        base = lax.rem(w0 + ci * CHUNK, T)

        @plsc.parallel_loop(0, CHUNK, step=1, unroll=4)
        def _(r):
            tt = lax.rem(base + r, T)
            for k in range(KD):
                sl = pl.ds(k * LANES, LANES)
                outb[ob][r, sl] = rows_v[b, r, sl] + pat_v[tt, sl]

    idx_start(0, 0)
    idx_wait(0, 0)
    for b in range(NBUF):
        gather_start(b, 0)

    def phase(g, pv):
        pn = (pv + 1) % 2

        @pl.when(g < NGRP - 1)
        def _():
            idx_start(g + 1, pn)

        for b in range(NBUF):
            ci = NBUF * g + b
            ob = b % 2
            gather_wait(b)
            if b < 2:
                @pl.when(g > 0)
                def _():
                    out_wait(ci - 2, ob)
            else:
                out_wait(ci - 2, ob)
            extract_add(ci, b, ob)
            out_start(ci, ob)

            @pl.when(g < NGRP - 1)
            def _():
                if b == 0:
                    idx_wait(g + 1, pn)
                gather_start(b, pn)

    def g_body(go, _):
        phase(2 * go, 0)
        phase(2 * go + 1, 1)
        return ()

    lax.fori_loop(0, NGRP // 2, g_body, ())
    out_wait(NCHUNK - 2, 0)
    out_wait(NCHUNK - 1, 1)


@jax.jit
def _embed(idx1d, tokp, posp):
    mesh = plsc.VectorSubcoreMesh(core_axis_name="c", subcore_axis_name="s")
    f = pl.kernel(
        _body,
        mesh=mesh,
        out_type=jax.ShapeDtypeStruct((FLAT, D), jnp.float32),
        scratch_types=[
            pltpu.VMEM((GROUP,), jnp.int32),
            pltpu.VMEM((GROUP,), jnp.int32),
            pltpu.VMEM((NBUF, CHUNK, 2 * D), jnp.float32),
            pltpu.VMEM((CHUNK, D), jnp.float32),
            pltpu.VMEM((CHUNK, D), jnp.float32),
            pltpu.VMEM((T, 2 * D), jnp.float32),
        ] + [pltpu.SemaphoreType.DMA] * 8,
        compiler_params=pltpu.CompilerParams(use_tc_tiling_on_sc=True),
    )
    return f(idx1d, tokp, posp)


def kernel(idx, token_embedding_table, position_embedding_table):
    idx1d = idx.astype(jnp.int32).reshape(FLAT)
    tokp = jnp.pad(token_embedding_table, ((0, 0), (0, D)))
    posp = jnp.pad(position_embedding_table, ((0, 0), (0, D)))
    out = _embed(idx1d, tokp, posp)
    return out.reshape(B, T, D)
